# baseline (speedup 1.0000x reference)
import functools

import jax
import jax.numpy as jnp
from jax import lax
from jax.experimental import pallas as pl
from jax.experimental.pallas import tpu as pltpu
from jax.experimental.pallas import tpu_sc as plsc

N_FACTORS = 32
BATCH = 16384
NC = 2
NS = 16
NW = NC * NS
BPW = BATCH // NW
CHUNK = 128
NCH = BPW // CHUNK
LANES = 16

_MESH = plsc.VectorSubcoreMesh(core_axis_name="c", subcore_axis_name="s")


def _gather_u_body(user_r, uf_r, out_r, uidx, rows, sem):
    wid = lax.axis_index("s") * NC + lax.axis_index("c")
    pltpu.sync_copy(user_r.at[wid], uidx)
    for j in range(NCH):
        pltpu.async_copy(uf_r.at[uidx.at[j]], rows, sem).wait()
        pltpu.sync_copy(rows, out_r.at[wid].at[pl.ds(j * CHUNK, CHUNK)])


_gather_u = functools.partial(
    pl.kernel,
    mesh=_MESH,
    out_type=jax.ShapeDtypeStruct((NW, BPW, N_FACTORS), jnp.float32),
    scratch_types=[
        pltpu.VMEM((NCH, CHUNK), jnp.int32),
        pltpu.VMEM((CHUNK, N_FACTORS), jnp.float32),
        pltpu.SemaphoreType.DMA,
    ],
    compiler_params=pltpu.CompilerParams(
        needs_layout_passes=False,
        use_tc_tiling_on_sc=False,
    ),
)(_gather_u_body)


def _dot_i_body(item_r, if_r, urows_r, out_r, iidx, uvm, irows, outv, sem):
    wid = lax.axis_index("s") * NC + lax.axis_index("c")
    pltpu.sync_copy(item_r.at[wid], iidx)
    pltpu.sync_copy(urows_r.at[wid], uvm)
    lane = lax.iota(jnp.int32, LANES)
    for j in range(NCH):
        def issue(p0, c):
            iv = iidx[j, pl.ds(p0 * LANES, LANES)]
            for q in range(LANES):
                p = p0 * LANES + q
                pltpu.async_copy(if_r.at[pl.ds(iv[q], 1)],
                                 irows.at[pl.ds(p, 1)], sem)
            return c

        lax.fori_loop(0, CHUNK // LANES, issue, 0)
        pltpu.make_async_copy(if_r.at[pl.ds(0, CHUNK)], irows, sem).wait()

        def group(gg, c):
            o = gg * LANES
            acc = jnp.zeros((LANES,), jnp.float32)
            for r in range(LANES):
                row = o + r
                urow = j * CHUNK + row
                s0 = uvm[urow, pl.ds(0, LANES)] * irows[row, pl.ds(0, LANES)]
                s1 = uvm[urow, pl.ds(LANES, LANES)] * irows[row, pl.ds(LANES, LANES)]
                tot = jnp.sum(s0 + s1)
                acc = jnp.where(lane == r, tot, acc)
            outv[pl.ds(j * CHUNK + o, LANES)] = acc
            return c

        lax.fori_loop(0, CHUNK // LANES, group, 0)

    pltpu.sync_copy(outv, out_r.at[wid])


_dot_i = functools.partial(
    pl.kernel,
    mesh=_MESH,
    out_type=jax.ShapeDtypeStruct((NW, BPW), jnp.float32),
    scratch_types=[
        pltpu.VMEM((NCH, CHUNK), jnp.int32),
        pltpu.VMEM((BPW, N_FACTORS), jnp.float32),
        pltpu.VMEM((CHUNK, N_FACTORS), jnp.float32),
        pltpu.VMEM((BPW,), jnp.float32),
        pltpu.SemaphoreType.DMA,
    ],
    compiler_params=pltpu.CompilerParams(needs_layout_passes=False),
)(_dot_i_body)


def kernel(user, item, user_factors, item_factors):
    u = user.astype(jnp.int32).reshape(NW, NCH, CHUNK)
    i = item.astype(jnp.int32).reshape(NW, NCH, CHUNK)
    urows = _gather_u(u, user_factors)
    out = _dot_i(i, item_factors, urows)
    return out.reshape(BATCH)


# speedup vs baseline: 1.3561x; 1.3561x over previous
import functools

import jax
import jax.numpy as jnp
from jax import lax
from jax.experimental import pallas as pl
from jax.experimental.pallas import tpu as pltpu
from jax.experimental.pallas import tpu_sc as plsc

N_FACTORS = 32
BATCH = 16384
NC = 2
NS = 16
NW = NC * NS
BPW = BATCH // NW
CHUNK = 128
NCH = BPW // CHUNK
LANES = 16


def _mf_body(user_r, item_r, uf_r, if_r, out_r,
             uidx, iidx, urows, irows, outv, sem):
    wid = lax.axis_index("s") * NC + lax.axis_index("c")

    pltpu.sync_copy(user_r.at[wid], uidx)
    pltpu.sync_copy(item_r.at[wid], iidx)

    lane = lax.iota(jnp.int32, LANES)

    for j in range(NCH):
        def issue(p0, c):
            uv = uidx[j, pl.ds(p0 * LANES, LANES)]
            iv = iidx[j, pl.ds(p0 * LANES, LANES)]
            for q in range(LANES):
                p = p0 * LANES + q
                pltpu.async_copy(uf_r.at[pl.ds(uv[q], 1)],
                                 urows.at[pl.ds(p, 1)], sem)
                pltpu.async_copy(if_r.at[pl.ds(iv[q], 1)],
                                 irows.at[pl.ds(p, 1)], sem)
            return c

        lax.fori_loop(0, CHUNK // LANES, issue, 0)

        pltpu.make_async_copy(uf_r.at[pl.ds(0, CHUNK)], urows, sem).wait()
        pltpu.make_async_copy(if_r.at[pl.ds(0, CHUNK)], irows, sem).wait()

        def group(gg, c):
            o = gg * LANES
            acc = jnp.zeros((LANES,), jnp.float32)
            for r in range(LANES):
                row = o + r
                s0 = urows[row, pl.ds(0, LANES)] * irows[row, pl.ds(0, LANES)]
                s1 = urows[row, pl.ds(LANES, LANES)] * irows[row, pl.ds(LANES, LANES)]
                tot = jnp.sum(s0 + s1)
                acc = jnp.where(lane == r, tot, acc)
            outv[pl.ds(j * CHUNK + o, LANES)] = acc
            return c

        lax.fori_loop(0, CHUNK // LANES, group, 0)

    pltpu.sync_copy(outv, out_r.at[wid])


_mf = functools.partial(
    pl.kernel,
    mesh=plsc.VectorSubcoreMesh(core_axis_name="c", subcore_axis_name="s"),
    out_type=jax.ShapeDtypeStruct((NW, BPW), jnp.float32),
    scratch_types=[
        pltpu.VMEM((NCH, CHUNK), jnp.int32),
        pltpu.VMEM((NCH, CHUNK), jnp.int32),
        pltpu.VMEM((CHUNK, N_FACTORS), jnp.float32),
        pltpu.VMEM((CHUNK, N_FACTORS), jnp.float32),
        pltpu.VMEM((BPW,), jnp.float32),
        pltpu.SemaphoreType.DMA,
    ],
    compiler_params=pltpu.CompilerParams(needs_layout_passes=False),
)(_mf_body)


def kernel(user, item, user_factors, item_factors):
    u = user.astype(jnp.int32).reshape(NW, NCH, CHUNK)
    i = item.astype(jnp.int32).reshape(NW, NCH, CHUNK)
    out = _mf(u, i, user_factors, item_factors)
    return out.reshape(BATCH)
